# baseline (device time: 72376 ns/iter reference)
import jax
import jax.numpy as jnp
from jax import lax
from jax.experimental import pallas as pl
from jax.experimental.pallas import tpu as pltpu

C = 8
CL = 4


def kernel(x):
    m, n = x.shape
    no = n // 2
    M = 2 * m
    hm = m // 2
    rpc = hm // C
    lrpc = m // CL

    def body(x_ref, out_ref, xs, sb, rb, xl, lb,
             lxs, lxl, lor, lol, s1, r1, s2, r2):
        my_x = lax.axis_index("x")
        my_y = lax.axis_index("y")
        other_x = 1 - my_x
        other_y = 1 - my_y

        barrier = pltpu.get_barrier_semaphore()
        pl.semaphore_signal(
            barrier, inc=1,
            device_id=(other_x, my_y), device_id_type=pl.DeviceIdType.MESH,
        )
        pl.semaphore_signal(
            barrier, inc=1,
            device_id=(my_x, other_y), device_id_type=pl.DeviceIdType.MESH,
        )
        pl.semaphore_wait(barrier, 2)

        def load_xs(c):
            return pltpu.make_async_copy(
                x_ref.at[pl.ds(my_y * hm + c * rpc, rpc), pl.ds(other_x * no, no)],
                xs.at[pl.ds(c * rpc, rpc), :],
                lxs.at[c],
            )

        def load_xl(c):
            return pltpu.make_async_copy(
                x_ref.at[pl.ds(c * lrpc, lrpc), pl.ds(my_x * no, no)],
                xl.at[pl.ds(c * lrpc, lrpc), :],
                lxl.at[c],
            )

        def store_local(c):
            return pltpu.make_async_copy(
                lb.at[pl.ds(c * lrpc, lrpc), :],
                out_ref.at[pl.ds(my_x * m + c * lrpc, lrpc), :],
                lol.at[c],
            )

        def store_recv(c):
            return pltpu.make_async_copy(
                rb.at[pl.ds(c * rpc, rpc), :],
                out_ref.at[pl.ds(other_x * m + my_y * hm + c * rpc, rpc), :],
                lor.at[c],
            )

        def rdma1(c):
            return pltpu.make_async_remote_copy(
                src_ref=sb.at[pl.ds(c * rpc, rpc), :],
                dst_ref=rb.at[pl.ds(c * rpc, rpc), :],
                send_sem=s1.at[c],
                recv_sem=r1.at[c],
                device_id=(other_x, my_y),
                device_id_type=pl.DeviceIdType.MESH,
            )

        def rdma2(c):
            return pltpu.make_async_remote_copy(
                src_ref=rb.at[pl.ds(c * rpc, rpc), :],
                dst_ref=out_ref.at[pl.ds(other_x * m + my_y * hm + c * rpc, rpc), :],
                send_sem=s2.at[c],
                recv_sem=r2.at[c],
                device_id=(my_x, other_y),
                device_id_type=pl.DeviceIdType.MESH,
            )

        def recv2(c):
            return pltpu.make_async_remote_copy(
                src_ref=rb.at[pl.ds(c * rpc, rpc), :],
                dst_ref=out_ref.at[pl.ds(other_x * m + other_y * hm + c * rpc, rpc), :],
                send_sem=s2.at[c],
                recv_sem=r2.at[c],
                device_id=(my_x, other_y),
                device_id_type=pl.DeviceIdType.MESH,
            )

        for c in range(C):
            load_xs(c).start()
        for c in range(CL):
            load_xl(c).start()

        for c in range(C):
            load_xs(c).wait()
            sb[pl.ds(c * rpc, rpc), :] = (
                xs[pl.ds(c * rpc, rpc), :].astype(jnp.bfloat16)
            )
            rdma1(c).start()

        for c in range(CL):
            load_xl(c).wait()
            lb[pl.ds(c * lrpc, lrpc), :] = (
                xl[pl.ds(c * lrpc, lrpc), :].astype(jnp.bfloat16)
            )
            store_local(c).start()

        for c in range(C):
            rdma1(c).wait_recv()
            rdma2(c).start()
            store_recv(c).start()

        for c in range(C):
            recv2(c).wait_recv()
        for c in range(C):
            store_recv(c).wait()
            rdma1(c).wait_send()
            rdma2(c).wait_send()
        for c in range(CL):
            store_local(c).wait()

    return pl.pallas_call(
        body,
        out_shape=jax.ShapeDtypeStruct((M, no), jnp.bfloat16),
        in_specs=[pl.BlockSpec(memory_space=pl.ANY)],
        out_specs=pl.BlockSpec(memory_space=pl.ANY),
        scratch_shapes=[
            pltpu.VMEM((hm, no), jnp.float32),
            pltpu.VMEM((hm, no), jnp.bfloat16),
            pltpu.VMEM((hm, no), jnp.bfloat16),
            pltpu.VMEM((m, no), jnp.float32),
            pltpu.VMEM((m, no), jnp.bfloat16),
            pltpu.SemaphoreType.DMA((C,)),
            pltpu.SemaphoreType.DMA((CL,)),
            pltpu.SemaphoreType.DMA((C,)),
            pltpu.SemaphoreType.DMA((CL,)),
            pltpu.SemaphoreType.DMA((C,)),
            pltpu.SemaphoreType.DMA((C,)),
            pltpu.SemaphoreType.DMA((C,)),
            pltpu.SemaphoreType.DMA((C,)),
        ],
        compiler_params=pltpu.CompilerParams(
            collective_id=0,
            vmem_limit_bytes=100 * 1024 * 1024,
        ),
    )(x)


# device time: 70321 ns/iter; 1.0292x vs baseline; 1.0292x over previous
import jax
import jax.numpy as jnp
from jax import lax
from jax.experimental import pallas as pl
from jax.experimental.pallas import tpu as pltpu

C = 16
CL = 4


def kernel(x):
    m, n = x.shape
    no = n // 2
    M = 2 * m
    hm = m // 2
    SZ = [hm // C] * C
    assert sum(SZ) == hm and len(SZ) == C
    OFF = [sum(SZ[:i]) for i in range(C)]
    lrpc = m // CL

    def body(x_ref, out_ref, xs, sb, rb, xl, lb,
             lxs, lxl, lor, lol, s1, r1, s2, r2):
        my_x = lax.axis_index("x")
        my_y = lax.axis_index("y")
        other_x = 1 - my_x
        other_y = 1 - my_y

        barrier = pltpu.get_barrier_semaphore()
        pl.semaphore_signal(
            barrier, inc=1,
            device_id=(other_x, my_y), device_id_type=pl.DeviceIdType.MESH,
        )
        pl.semaphore_signal(
            barrier, inc=1,
            device_id=(my_x, other_y), device_id_type=pl.DeviceIdType.MESH,
        )
        pl.semaphore_wait(barrier, 2)

        def load_xs(c):
            return pltpu.make_async_copy(
                x_ref.at[pl.ds(my_y * hm + OFF[c], SZ[c]), pl.ds(other_x * no, no)],
                xs.at[pl.ds(OFF[c], SZ[c]), :],
                lxs.at[c],
            )

        def load_xl(c):
            return pltpu.make_async_copy(
                x_ref.at[pl.ds(c * lrpc, lrpc), pl.ds(my_x * no, no)],
                xl.at[pl.ds(c * lrpc, lrpc), :],
                lxl.at[c],
            )

        def store_local(c):
            return pltpu.make_async_copy(
                lb.at[pl.ds(c * lrpc, lrpc), :],
                out_ref.at[pl.ds(my_x * m + c * lrpc, lrpc), :],
                lol.at[c],
            )

        def store_recv(c):
            return pltpu.make_async_copy(
                rb.at[pl.ds(OFF[c], SZ[c]), :],
                out_ref.at[pl.ds(other_x * m + my_y * hm + OFF[c], SZ[c]), :],
                lor.at[c],
            )

        def rdma1(c):
            return pltpu.make_async_remote_copy(
                src_ref=sb.at[pl.ds(OFF[c], SZ[c]), :],
                dst_ref=rb.at[pl.ds(OFF[c], SZ[c]), :],
                send_sem=s1.at[c],
                recv_sem=r1.at[c],
                device_id=(other_x, my_y),
                device_id_type=pl.DeviceIdType.MESH,
            )

        def rdma2(c):
            return pltpu.make_async_remote_copy(
                src_ref=rb.at[pl.ds(OFF[c], SZ[c]), :],
                dst_ref=out_ref.at[pl.ds(other_x * m + my_y * hm + OFF[c], SZ[c]), :],
                send_sem=s2.at[c],
                recv_sem=r2.at[c],
                device_id=(my_x, other_y),
                device_id_type=pl.DeviceIdType.MESH,
            )

        def recv2(c):
            return pltpu.make_async_remote_copy(
                src_ref=rb.at[pl.ds(OFF[c], SZ[c]), :],
                dst_ref=out_ref.at[pl.ds(other_x * m + other_y * hm + OFF[c], SZ[c]), :],
                send_sem=s2.at[c],
                recv_sem=r2.at[c],
                device_id=(my_x, other_y),
                device_id_type=pl.DeviceIdType.MESH,
            )

        for c in range(C):
            load_xs(c).start()
        for c in range(CL):
            load_xl(c).start()

        for c in range(C):
            load_xs(c).wait()
            sb[pl.ds(OFF[c], SZ[c]), :] = (
                xs[pl.ds(OFF[c], SZ[c]), :].astype(jnp.bfloat16)
            )
            rdma1(c).start()

        for c in range(C):
            rdma1(c).wait_recv()
            rdma2(c).start()
            store_recv(c).start()
            if c % (C // CL) == C // CL - 1:
                lc = c // (C // CL)
                load_xl(lc).wait()
                lb[pl.ds(lc * lrpc, lrpc), :] = (
                    xl[pl.ds(lc * lrpc, lrpc), :].astype(jnp.bfloat16)
                )
                store_local(lc).start()

        for c in range(C):
            recv2(c).wait_recv()
        for c in range(C):
            store_recv(c).wait()
            rdma1(c).wait_send()
            rdma2(c).wait_send()
        for c in range(CL):
            store_local(c).wait()

    return pl.pallas_call(
        body,
        out_shape=jax.ShapeDtypeStruct((M, no), jnp.bfloat16),
        in_specs=[pl.BlockSpec(memory_space=pl.ANY)],
        out_specs=pl.BlockSpec(memory_space=pl.ANY),
        scratch_shapes=[
            pltpu.VMEM((hm, no), jnp.float32),
            pltpu.VMEM((hm, no), jnp.bfloat16),
            pltpu.VMEM((hm, no), jnp.bfloat16),
            pltpu.VMEM((m, no), jnp.float32),
            pltpu.VMEM((m, no), jnp.bfloat16),
            pltpu.SemaphoreType.DMA((C,)),
            pltpu.SemaphoreType.DMA((CL,)),
            pltpu.SemaphoreType.DMA((C,)),
            pltpu.SemaphoreType.DMA((CL,)),
            pltpu.SemaphoreType.DMA((C,)),
            pltpu.SemaphoreType.DMA((C,)),
            pltpu.SemaphoreType.DMA((C,)),
            pltpu.SemaphoreType.DMA((C,)),
        ],
        compiler_params=pltpu.CompilerParams(
            collective_id=0,
            vmem_limit_bytes=100 * 1024 * 1024,
        ),
    )(x)
